# baseline (speedup 1.0000x reference)
import functools

import jax
import jax.numpy as jnp
from jax import lax
from jax.experimental import pallas as pl
from jax.experimental.pallas import tpu as pltpu
from jax.experimental.pallas import tpu_sc as plsc

VOCAB = 1000000
D_MODEL = 64
B_ROWS = 4096
T_COLS = 200
B_TOTAL = B_ROWS * T_COLS

_info = plsc.get_sparse_core_info()
NC = _info.num_cores
NS = _info.num_subcores
NW = NC * NS
B_PER_W = B_TOTAL // NW

CHUNK = 128
N_CHUNKS = B_PER_W // CHUNK


def _make_gather():
    mesh = plsc.VectorSubcoreMesh(core_axis_name="c", subcore_axis_name="s")

    @functools.partial(
        pl.kernel,
        mesh=mesh,
        out_type=jax.ShapeDtypeStruct((B_TOTAL, D_MODEL), jnp.float32),
        scratch_types=[
            pltpu.VMEM((CHUNK,), jnp.int32),
            pltpu.VMEM((CHUNK, D_MODEL), jnp.float32),
            pltpu.SemaphoreType.DMA,
        ],
        compiler_params=pltpu.CompilerParams(use_tc_tiling_on_sc=False),
    )
    def gather_kernel(idx_hbm, table_hbm, out_hbm, idx_v, rows_v, sem):
        wid = lax.axis_index("s") * NC + lax.axis_index("c")
        base = wid * B_PER_W

        def body(c, carry):
            off = base + c * CHUNK
            pltpu.sync_copy(idx_hbm.at[pl.ds(off, CHUNK)], idx_v)
            pltpu.async_copy(table_hbm.at[idx_v], rows_v, sem).wait()
            pltpu.sync_copy(rows_v, out_hbm.at[pl.ds(off, CHUNK)])
            return carry

        lax.fori_loop(0, N_CHUNKS, body, 0, unroll=False)

    return gather_kernel


_gather = _make_gather()


def kernel(token_ids, table):
    idx_flat = token_ids.reshape(B_TOTAL)
    out = _gather(idx_flat, table)
    return out.reshape(B_ROWS, T_COLS, D_MODEL)


# speedup vs baseline: 1.1956x; 1.1956x over previous
import functools

import jax
import jax.numpy as jnp
from jax import lax
from jax.experimental import pallas as pl
from jax.experimental.pallas import tpu as pltpu
from jax.experimental.pallas import tpu_sc as plsc

VOCAB = 1000000
D_MODEL = 64
B_ROWS = 4096
T_COLS = 200
B_TOTAL = B_ROWS * T_COLS

_info = plsc.get_sparse_core_info()
NC = _info.num_cores
NS = _info.num_subcores
NW = NC * NS
B_PER_W = B_TOTAL // NW

CHUNK = 128
N_CHUNKS = B_PER_W // CHUNK
N_BUF = 4
N_GROUPS = N_CHUNKS // N_BUF


def _make_gather():
    mesh = plsc.VectorSubcoreMesh(core_axis_name="c", subcore_axis_name="s")

    @functools.partial(
        pl.kernel,
        mesh=mesh,
        out_type=jax.ShapeDtypeStruct((NW, N_CHUNKS, CHUNK, D_MODEL), jnp.float32),
        scratch_types=[
            pltpu.VMEM((N_CHUNKS, CHUNK), jnp.int32),
            pltpu.VMEM((N_BUF, CHUNK, D_MODEL), jnp.float32),
            pltpu.SemaphoreType.DMA,
            pltpu.SemaphoreType.DMA((N_BUF,)),
            pltpu.SemaphoreType.DMA((N_BUF,)),
        ],
        compiler_params=pltpu.CompilerParams(use_tc_tiling_on_sc=False),
    )
    def gather_kernel(idx_hbm, table_hbm, out_hbm, idx_v, bufs, isem, gsem, wsem):
        wid = lax.axis_index("s") * NC + lax.axis_index("c")
        pltpu.sync_copy(idx_hbm.at[wid], idx_v)

        def gather_start(b, c):
            pltpu.async_copy(table_hbm.at[idx_v.at[c]], bufs.at[b], gsem.at[b])

        def gather_wait(b):
            pltpu.make_async_copy(table_hbm.at[idx_v.at[0]], bufs.at[b],
                                  gsem.at[b]).wait()

        def wb_start(b, c):
            pltpu.async_copy(bufs.at[b], out_hbm.at[wid, c], wsem.at[b])

        def wb_wait(b):
            pltpu.make_async_copy(bufs.at[b], out_hbm.at[wid, 0],
                                  wsem.at[b]).wait()

        for b in range(N_BUF):
            gather_start(b, b)

        def outer(g, carry):
            for b in range(N_BUF):
                gather_wait(b)
                wb_start(b, g * N_BUF + b)
            for b in range(N_BUF):
                wb_wait(b)

                @pl.when(g + 1 < N_GROUPS)
                def _():
                    gather_start(b, (g + 1) * N_BUF + b)
            return carry

        lax.fori_loop(0, N_GROUPS, outer, 0, unroll=False)

    return gather_kernel


_gather = _make_gather()


def kernel(token_ids, table):
    idx = token_ids.reshape(NW, N_CHUNKS, CHUNK)
    out = _gather(idx, table)
    return out.reshape(B_ROWS, T_COLS, D_MODEL)
